# Initial kernel scaffold; baseline (speedup 1.0000x reference)
import jax
import jax.numpy as jnp
from jax.experimental import pallas as pl


def kernel(features, roi):
    raise NotImplementedError("write your pallas kernel here")


# speedup vs baseline: 21.4820x; 21.4820x over previous
import jax
import jax.numpy as jnp
from jax import lax
from jax.experimental import pallas as pl
from jax.experimental.pallas import tpu as pltpu

_POOL = 7
_NREG = 32
_THR = 0.4
_NPAD = 1024
_N = 1000
_WIN = 64
_WINW = 72


def _nms_clip_kernel(rois_t_ref, rois_ref, out_ref):
    xs = rois_t_ref[0, 0:1, :]
    ys = rois_t_ref[0, 1:2, :]
    ws = rois_t_ref[0, 2:3, :]
    hs = rois_t_ref[0, 3:4, :]
    iota = lax.broadcasted_iota(jnp.int32, (1, _NPAD), 1)
    keep0 = jnp.where(iota < _N, 1.0, 0.0).astype(jnp.float32)

    def body(p, keep):
        oh = iota == p
        kp = jnp.max(jnp.where(oh, keep, 0.0)) > 0.0

        def do_suppress(k):
            xp = jnp.sum(jnp.where(oh, xs, 0.0))
            yp = jnp.sum(jnp.where(oh, ys, 0.0))
            wp = jnp.sum(jnp.where(oh, ws, 0.0))
            hp = jnp.sum(jnp.where(oh, hs, 0.0))
            x1 = jnp.maximum(xp, xs)
            y1 = jnp.maximum(yp, ys)
            x2 = jnp.minimum(xp + wp, xs + ws)
            y2 = jnp.minimum(yp + hp, ys + hs)
            inter = jnp.maximum(0.0, x2 - x1) * jnp.maximum(0.0, y2 - y1)
            union = wp * hp + ws * hs - inter
            iou = inter / jnp.maximum(union, 1e-9)
            suppress = jnp.logical_and(iou > _THR, iota > p)
            return jnp.where(suppress, 0.0, k)

        return lax.cond(kp, do_suppress, lambda k: k, keep)

    keep = lax.fori_loop(0, _N - 1, body, keep0)

    kiota = lax.broadcasted_iota(jnp.int32, (_NREG, 1), 0)

    def pick(k, carry):
        rem, idxc = carry
        cand = jnp.where(rem > 0.0, iota, _NPAD * 4)
        m = jnp.min(cand)
        idx_k = jnp.minimum(m, _N - 1)
        rem = jnp.where(iota == m, 0.0, rem)
        idxc = jnp.where(kiota == k, idx_k, idxc)
        return rem, idxc

    _, idxc = lax.fori_loop(
        0, _NREG, pick, (keep, jnp.zeros((_NREG, 1), jnp.int32))
    )

    iota_b = lax.broadcasted_iota(jnp.int32, (_NREG, _NPAD), 1)
    onehot = (idxc == iota_b).astype(jnp.float32)
    boxes = jnp.dot(
        onehot,
        rois_ref[0],
        preferred_element_type=jnp.float32,
        precision=lax.Precision.HIGHEST,
    )

    x = boxes[:, 0:1]
    y = boxes[:, 1:2]
    w = boxes[:, 2:3]
    h = boxes[:, 3:4]
    fs = 224
    c0 = jnp.maximum(0, x.astype(jnp.int32))
    c1 = jnp.maximum(0, y.astype(jnp.int32))
    c2 = jnp.minimum(fs, (x + w).astype(jnp.int32))
    c3 = jnp.minimum(fs, (y + h).astype(jnp.int32))

    def fix_dim(cmin, cmax):
        pad = _POOL - (cmax - cmin)
        fix_min = cmin < pad // 2
        fix_max = fs - cmax < (1 + pad) // 2
        pos = pad > 0
        sym = jnp.logical_and(pos, jnp.logical_not(jnp.logical_or(fix_min, fix_max)))
        ncmin = jnp.where(sym, cmin - pad // 2, cmin)
        ncmax = jnp.where(sym, cmax + (1 + pad) // 2, cmax)
        m1 = jnp.logical_and(pos, fix_min)
        ncmin = jnp.where(m1, 0, ncmin)
        ncmax = jnp.where(m1, _POOL, ncmax)
        m2 = jnp.logical_and(pos, fix_max)
        ncmin = jnp.where(m2, fs - _POOL, ncmin)
        ncmax = jnp.where(m2, fs, ncmax)
        return ncmin, ncmax

    c0, c2 = fix_dim(c0, c2)
    c1, c3 = fix_dim(c1, c3)
    res = jnp.concatenate([c0, c1, c2 - c0, c3 - c1], axis=1)
    out_ref[...] = res[None, :, :]


def _pool_kernel(coords_ref, feat_ref, out_ref, slab_ref, sem):
    b = pl.program_id(0)
    r = pl.program_id(1)
    x0 = coords_ref[b, r, 0]
    y0 = coords_ref[b, r, 1]
    w = coords_ref[b, r, 2]
    h = coords_ref[b, r, 3]
    H = feat_ref.shape[1]
    W = feat_ref.shape[2]
    ysl = jnp.minimum(y0, H - _WIN)
    xsl = jnp.minimum((x0 // 8) * 8, W - _WINW)
    copy = pltpu.make_async_copy(
        feat_ref.at[b, pl.ds(ysl, _WIN), pl.ds(xsl, _WINW), :], slab_ref, sem
    )
    copy.start()
    copy.wait()
    slab = slab_ref[...]

    oy = y0 - ysl
    ox = x0 - xsl
    hstep = h // _POOL
    wstep = w // _POOL
    neg = jnp.float32(-jnp.inf)
    riota = lax.broadcasted_iota(jnp.int32, (_WIN, 1, 1), 0)
    ciota = lax.broadcasted_iota(jnp.int32, (_WINW, 1), 0)
    for i in range(_POOL):
        ya = oy + i * hstep
        yb = oy + (i + 1) * hstep if i + 1 < _POOL else oy + h
        rm = jnp.logical_and(riota >= ya, riota < yb)
        rowmax = jnp.max(jnp.where(rm, slab, neg), axis=0)
        for j in range(_POOL):
            xa = ox + j * wstep
            xb = ox + (j + 1) * wstep if j + 1 < _POOL else ox + w
            cm = jnp.logical_and(ciota >= xa, ciota < xb)
            cell = jnp.max(jnp.where(cm, rowmax, neg), axis=0)
            out_ref[0, 0, i, j, :] = cell


def kernel(features, roi):
    B, N = roi.shape[0], roi.shape[1]
    H, W, C = features.shape[1], features.shape[2], features.shape[3]
    rois_p = jnp.pad(roi, ((0, 0), (0, _NPAD - N), (0, 0)))
    rois_t = jnp.transpose(rois_p, (0, 2, 1))

    coords = pl.pallas_call(
        _nms_clip_kernel,
        grid=(B,),
        in_specs=[
            pl.BlockSpec((1, 4, _NPAD), lambda b: (b, 0, 0)),
            pl.BlockSpec((1, _NPAD, 4), lambda b: (b, 0, 0)),
        ],
        out_specs=pl.BlockSpec((1, _NREG, 4), lambda b: (b, 0, 0)),
        out_shape=jax.ShapeDtypeStruct((B, _NREG, 4), jnp.int32),
    )(rois_t, rois_p)

    out = pl.pallas_call(
        _pool_kernel,
        grid=(B, _NREG),
        in_specs=[
            pl.BlockSpec(memory_space=pltpu.SMEM),
            pl.BlockSpec(memory_space=pl.ANY),
        ],
        out_specs=pl.BlockSpec(
            (1, 1, _POOL, _POOL, C), lambda b, r: (b, r, 0, 0, 0)
        ),
        out_shape=jax.ShapeDtypeStruct((B, _NREG, _POOL, _POOL, C), jnp.float32),
        scratch_shapes=[
            pltpu.VMEM((_WIN, _WINW, C), jnp.float32),
            pltpu.SemaphoreType.DMA,
        ],
    )(coords, features)
    return out


# speedup vs baseline: 32.6939x; 1.5219x over previous
import jax
import jax.numpy as jnp
from jax import lax
from jax.experimental import pallas as pl
from jax.experimental.pallas import tpu as pltpu

_POOL = 7
_NREG = 32
_THR = 0.4
_NPAD = 1024
_N = 1000
_WIN = 64
_WINW = 72


def _nms_clip_kernel(rois_t_ref, rois_ref, out_ref):
    xs = rois_t_ref[0, 0:1, :]
    ys = rois_t_ref[0, 1:2, :]
    ws = rois_t_ref[0, 2:3, :]
    hs = rois_t_ref[0, 3:4, :]
    iota = lax.broadcasted_iota(jnp.int32, (1, _NPAD), 1)
    keep0 = jnp.where(iota < _N, 1.0, 0.0).astype(jnp.float32)

    def body(p, keep):
        oh = iota == p
        kp = jnp.max(jnp.where(oh, keep, 0.0)) > 0.0

        def do_suppress(k):
            xp = jnp.sum(jnp.where(oh, xs, 0.0))
            yp = jnp.sum(jnp.where(oh, ys, 0.0))
            wp = jnp.sum(jnp.where(oh, ws, 0.0))
            hp = jnp.sum(jnp.where(oh, hs, 0.0))
            x1 = jnp.maximum(xp, xs)
            y1 = jnp.maximum(yp, ys)
            x2 = jnp.minimum(xp + wp, xs + ws)
            y2 = jnp.minimum(yp + hp, ys + hs)
            inter = jnp.maximum(0.0, x2 - x1) * jnp.maximum(0.0, y2 - y1)
            union = wp * hp + ws * hs - inter
            iou = inter / jnp.maximum(union, 1e-9)
            suppress = jnp.logical_and(iou > _THR, iota > p)
            return jnp.where(suppress, 0.0, k)

        return lax.cond(kp, do_suppress, lambda k: k, keep)

    keep = lax.fori_loop(0, _N - 1, body, keep0)

    kiota = lax.broadcasted_iota(jnp.int32, (_NREG, 1), 0)

    def pick(k, carry):
        rem, idxc = carry
        cand = jnp.where(rem > 0.0, iota, _NPAD * 4)
        m = jnp.min(cand)
        idx_k = jnp.minimum(m, _N - 1)
        rem = jnp.where(iota == m, 0.0, rem)
        idxc = jnp.where(kiota == k, idx_k, idxc)
        return rem, idxc

    _, idxc = lax.fori_loop(
        0, _NREG, pick, (keep, jnp.zeros((_NREG, 1), jnp.int32))
    )

    iota_b = lax.broadcasted_iota(jnp.int32, (_NREG, _NPAD), 1)
    onehot = (idxc == iota_b).astype(jnp.float32)
    boxes = jnp.dot(
        onehot,
        rois_ref[0],
        preferred_element_type=jnp.float32,
        precision=lax.Precision.HIGHEST,
    )

    x = boxes[:, 0:1]
    y = boxes[:, 1:2]
    w = boxes[:, 2:3]
    h = boxes[:, 3:4]
    fs = 224
    c0 = jnp.maximum(0, x.astype(jnp.int32))
    c1 = jnp.maximum(0, y.astype(jnp.int32))
    c2 = jnp.minimum(fs, (x + w).astype(jnp.int32))
    c3 = jnp.minimum(fs, (y + h).astype(jnp.int32))

    def fix_dim(cmin, cmax):
        pad = _POOL - (cmax - cmin)
        fix_min = cmin < pad // 2
        fix_max = fs - cmax < (1 + pad) // 2
        pos = pad > 0
        sym = jnp.logical_and(pos, jnp.logical_not(jnp.logical_or(fix_min, fix_max)))
        ncmin = jnp.where(sym, cmin - pad // 2, cmin)
        ncmax = jnp.where(sym, cmax + (1 + pad) // 2, cmax)
        m1 = jnp.logical_and(pos, fix_min)
        ncmin = jnp.where(m1, 0, ncmin)
        ncmax = jnp.where(m1, _POOL, ncmax)
        m2 = jnp.logical_and(pos, fix_max)
        ncmin = jnp.where(m2, fs - _POOL, ncmin)
        ncmax = jnp.where(m2, fs, ncmax)
        return ncmin, ncmax

    c0, c2 = fix_dim(c0, c2)
    c1, c3 = fix_dim(c1, c3)
    res = jnp.concatenate([c0, c1, c2 - c0, c3 - c1], axis=1)
    out_ref[...] = res[None, :, :]


_BIN = 16


def _pool_kernel(coords_ref, feat_ref, out_ref, slab_ref, sem_ref):
    b = pl.program_id(0)
    r = pl.program_id(1)
    H = feat_ref.shape[1]
    W = feat_ref.shape[2]
    g = b * _NREG + r
    total = pl.num_programs(0) * _NREG

    def issue(gg, buf):
        b2 = gg // _NREG
        r2 = gg % _NREG
        x = coords_ref[b2, r2, 0]
        y = coords_ref[b2, r2, 1]
        ysl2 = jnp.minimum(y, H - _WIN)
        xsl2 = jnp.minimum((x // 8) * 8, W - _WINW)
        pltpu.make_async_copy(
            feat_ref.at[b2, pl.ds(ysl2, _WIN), pl.ds(xsl2, _WINW), :],
            slab_ref.at[buf],
            sem_ref.at[buf],
        ).start()

    @pl.when(g == 0)
    def _():
        issue(0, 0)

    @pl.when(g + 1 < total)
    def _():
        issue(g + 1, (g + 1) % 2)

    buf = g % 2
    pltpu.make_async_copy(
        feat_ref.at[b, pl.ds(0, _WIN), pl.ds(0, _WINW), :],
        slab_ref.at[buf],
        sem_ref.at[buf],
    ).wait()

    x0 = coords_ref[b, r, 0]
    y0 = coords_ref[b, r, 1]
    w = coords_ref[b, r, 2]
    h = coords_ref[b, r, 3]
    ysl = jnp.minimum(y0, H - _WIN)
    xsl = jnp.minimum((x0 // 8) * 8, W - _WINW)
    oy = y0 - ysl
    ox = x0 - xsl
    hstep = h // _POOL
    wstep = w // _POOL
    neg = jnp.float32(-jnp.inf)
    riota = lax.broadcasted_iota(jnp.int32, (_BIN, 1, 1), 0)
    ciota = lax.broadcasted_iota(jnp.int32, (_WINW, 1), 0)
    for i in range(_POOL):
        ya = oy + i * hstep
        yb = oy + (i + 1) * hstep if i + 1 < _POOL else oy + h
        ya_c = jnp.minimum(ya, _WIN - _BIN)
        chunk = slab_ref[buf, pl.ds(ya_c, _BIN), :, :]
        absr = riota + ya_c
        rm = jnp.logical_and(absr >= ya, absr < yb)
        rowmax = jnp.max(jnp.where(rm, chunk, neg), axis=0)
        for j in range(_POOL):
            xa = ox + j * wstep
            xb = ox + (j + 1) * wstep if j + 1 < _POOL else ox + w
            cm = jnp.logical_and(ciota >= xa, ciota < xb)
            cell = jnp.max(jnp.where(cm, rowmax, neg), axis=0)
            out_ref[0, 0, i, j, :] = cell


def kernel(features, roi):
    B, N = roi.shape[0], roi.shape[1]
    H, W, C = features.shape[1], features.shape[2], features.shape[3]
    rois_p = jnp.pad(roi, ((0, 0), (0, _NPAD - N), (0, 0)))
    rois_t = jnp.transpose(rois_p, (0, 2, 1))

    coords = pl.pallas_call(
        _nms_clip_kernel,
        grid=(B,),
        in_specs=[
            pl.BlockSpec((1, 4, _NPAD), lambda b: (b, 0, 0)),
            pl.BlockSpec((1, _NPAD, 4), lambda b: (b, 0, 0)),
        ],
        out_specs=pl.BlockSpec((1, _NREG, 4), lambda b: (b, 0, 0)),
        out_shape=jax.ShapeDtypeStruct((B, _NREG, 4), jnp.int32),
    )(rois_t, rois_p)

    out = pl.pallas_call(
        _pool_kernel,
        grid=(B, _NREG),
        in_specs=[
            pl.BlockSpec(memory_space=pltpu.SMEM),
            pl.BlockSpec(memory_space=pl.ANY),
        ],
        out_specs=pl.BlockSpec(
            (1, 1, _POOL, _POOL, C), lambda b, r: (b, r, 0, 0, 0)
        ),
        out_shape=jax.ShapeDtypeStruct((B, _NREG, _POOL, _POOL, C), jnp.float32),
        scratch_shapes=[
            pltpu.VMEM((2, _WIN, _WINW, C), jnp.float32),
            pltpu.SemaphoreType.DMA((2,)),
        ],
    )(coords, features)
    return out


# speedup vs baseline: 34.2374x; 1.0472x over previous
import jax
import jax.numpy as jnp
from jax import lax
from jax.experimental import pallas as pl
from jax.experimental.pallas import tpu as pltpu

_POOL = 7
_NREG = 32
_THR = 0.4
_NPAD = 1024
_N = 1000
_WIN = 64
_WINW = 72


def _nms_clip_kernel(rois_q_ref, rois_ref, out_ref):
    xs = rois_q_ref[0, 0]
    ys = rois_q_ref[0, 1]
    ws = rois_q_ref[0, 2]
    hs = rois_q_ref[0, 3]
    iota = lax.broadcasted_iota(jnp.int32, (8, 128), 0) * 128 + lax.broadcasted_iota(
        jnp.int32, (8, 128), 1
    )
    keep0 = jnp.where(iota < _N, 1.0, 0.0).astype(jnp.float32)

    def body(p, keep):
        oh = iota == p
        kp = jnp.max(jnp.where(oh, keep, 0.0)) > 0.0

        def do_suppress(k):
            xp = jnp.sum(jnp.where(oh, xs, 0.0))
            yp = jnp.sum(jnp.where(oh, ys, 0.0))
            wp = jnp.sum(jnp.where(oh, ws, 0.0))
            hp = jnp.sum(jnp.where(oh, hs, 0.0))
            x1 = jnp.maximum(xp, xs)
            y1 = jnp.maximum(yp, ys)
            x2 = jnp.minimum(xp + wp, xs + ws)
            y2 = jnp.minimum(yp + hp, ys + hs)
            inter = jnp.maximum(0.0, x2 - x1) * jnp.maximum(0.0, y2 - y1)
            union = wp * hp + ws * hs - inter
            iou = inter / jnp.maximum(union, 1e-9)
            suppress = jnp.logical_and(iou > _THR, iota > p)
            return jnp.where(suppress, 0.0, k)

        return lax.cond(kp, do_suppress, lambda k: k, keep)

    keep = lax.fori_loop(0, _N - 1, body, keep0)

    kiota = lax.broadcasted_iota(jnp.int32, (_NREG, 1), 0)

    def pick(k, carry):
        rem, idxc = carry
        cand = jnp.where(rem > 0.0, iota, _NPAD * 4)
        m = jnp.min(cand)
        idx_k = jnp.minimum(m, _N - 1)
        rem = jnp.where(iota == m, 0.0, rem)
        idxc = jnp.where(kiota == k, idx_k, idxc)
        return rem, idxc

    _, idxc = lax.fori_loop(
        0, _NREG, pick, (keep, jnp.zeros((_NREG, 1), jnp.int32))
    )

    iota_b = lax.broadcasted_iota(jnp.int32, (_NREG, _NPAD), 1)
    onehot = (idxc == iota_b).astype(jnp.float32)
    boxes = jnp.dot(
        onehot,
        rois_ref[0],
        preferred_element_type=jnp.float32,
        precision=lax.Precision.HIGHEST,
    )

    x = boxes[:, 0:1]
    y = boxes[:, 1:2]
    w = boxes[:, 2:3]
    h = boxes[:, 3:4]
    fs = 224
    c0 = jnp.maximum(0, x.astype(jnp.int32))
    c1 = jnp.maximum(0, y.astype(jnp.int32))
    c2 = jnp.minimum(fs, (x + w).astype(jnp.int32))
    c3 = jnp.minimum(fs, (y + h).astype(jnp.int32))

    def fix_dim(cmin, cmax):
        pad = _POOL - (cmax - cmin)
        fix_min = cmin < pad // 2
        fix_max = fs - cmax < (1 + pad) // 2
        pos = pad > 0
        sym = jnp.logical_and(pos, jnp.logical_not(jnp.logical_or(fix_min, fix_max)))
        ncmin = jnp.where(sym, cmin - pad // 2, cmin)
        ncmax = jnp.where(sym, cmax + (1 + pad) // 2, cmax)
        m1 = jnp.logical_and(pos, fix_min)
        ncmin = jnp.where(m1, 0, ncmin)
        ncmax = jnp.where(m1, _POOL, ncmax)
        m2 = jnp.logical_and(pos, fix_max)
        ncmin = jnp.where(m2, fs - _POOL, ncmin)
        ncmax = jnp.where(m2, fs, ncmax)
        return ncmin, ncmax

    c0, c2 = fix_dim(c0, c2)
    c1, c3 = fix_dim(c1, c3)
    res = jnp.concatenate([c0, c1, c2 - c0, c3 - c1], axis=1)
    out_ref[...] = res[None, :, :]


_BIN = 16


def _pool_kernel(coords_ref, feat_ref, out_ref, slab_ref, sem_ref):
    b = pl.program_id(0)
    r = pl.program_id(1)
    H = feat_ref.shape[1]
    W = feat_ref.shape[2]
    g = b * _NREG + r
    total = pl.num_programs(0) * _NREG

    def issue(gg, buf):
        b2 = gg // _NREG
        r2 = gg % _NREG
        x = coords_ref[b2, r2, 0]
        y = coords_ref[b2, r2, 1]
        ysl2 = jnp.minimum(y, H - _WIN)
        xsl2 = jnp.minimum((x // 8) * 8, W - _WINW)
        pltpu.make_async_copy(
            feat_ref.at[b2, pl.ds(ysl2, _WIN), pl.ds(xsl2, _WINW), :],
            slab_ref.at[buf],
            sem_ref.at[buf],
        ).start()

    @pl.when(g == 0)
    def _():
        issue(0, 0)

    @pl.when(g + 1 < total)
    def _():
        issue(g + 1, (g + 1) % 2)

    buf = g % 2
    pltpu.make_async_copy(
        feat_ref.at[b, pl.ds(0, _WIN), pl.ds(0, _WINW), :],
        slab_ref.at[buf],
        sem_ref.at[buf],
    ).wait()

    x0 = coords_ref[b, r, 0]
    y0 = coords_ref[b, r, 1]
    w = coords_ref[b, r, 2]
    h = coords_ref[b, r, 3]
    ysl = jnp.minimum(y0, H - _WIN)
    xsl = jnp.minimum((x0 // 8) * 8, W - _WINW)
    oy = y0 - ysl
    ox = x0 - xsl
    hstep = h // _POOL
    wstep = w // _POOL
    neg = jnp.float32(-jnp.inf)
    riota = lax.broadcasted_iota(jnp.int32, (_BIN, 1, 1), 0)
    ciota = lax.broadcasted_iota(jnp.int32, (_WINW, 1), 0)
    for i in range(_POOL):
        ya = oy + i * hstep
        yb = oy + (i + 1) * hstep if i + 1 < _POOL else oy + h
        ya_c = jnp.minimum(ya, _WIN - _BIN)
        chunk = slab_ref[buf, pl.ds(ya_c, _BIN), :, :]
        absr = riota + ya_c
        rm = jnp.logical_and(absr >= ya, absr < yb)
        rowmax = jnp.max(jnp.where(rm, chunk, neg), axis=0)
        for j in range(_POOL):
            xa = ox + j * wstep
            xb = ox + (j + 1) * wstep if j + 1 < _POOL else ox + w
            cm = jnp.logical_and(ciota >= xa, ciota < xb)
            cell = jnp.max(jnp.where(cm, rowmax, neg), axis=0)
            out_ref[0, 0, i, j, :] = cell


def kernel(features, roi):
    B, N = roi.shape[0], roi.shape[1]
    H, W, C = features.shape[1], features.shape[2], features.shape[3]
    rois_p = jnp.pad(roi, ((0, 0), (0, _NPAD - N), (0, 0)))
    rois_q = jnp.transpose(rois_p, (0, 2, 1)).reshape(B, 4, 8, 128)

    coords = pl.pallas_call(
        _nms_clip_kernel,
        grid=(B,),
        in_specs=[
            pl.BlockSpec((1, 4, 8, 128), lambda b: (b, 0, 0, 0)),
            pl.BlockSpec((1, _NPAD, 4), lambda b: (b, 0, 0)),
        ],
        out_specs=pl.BlockSpec((1, _NREG, 4), lambda b: (b, 0, 0)),
        out_shape=jax.ShapeDtypeStruct((B, _NREG, 4), jnp.int32),
    )(rois_q, rois_p)

    out = pl.pallas_call(
        _pool_kernel,
        grid=(B, _NREG),
        in_specs=[
            pl.BlockSpec(memory_space=pltpu.SMEM),
            pl.BlockSpec(memory_space=pl.ANY),
        ],
        out_specs=pl.BlockSpec(
            (1, 1, _POOL, _POOL, C), lambda b, r: (b, r, 0, 0, 0)
        ),
        out_shape=jax.ShapeDtypeStruct((B, _NREG, _POOL, _POOL, C), jnp.float32),
        scratch_shapes=[
            pltpu.VMEM((2, _WIN, _WINW, C), jnp.float32),
            pltpu.SemaphoreType.DMA((2,)),
        ],
    )(coords, features)
    return out


# speedup vs baseline: 38.7244x; 1.1311x over previous
import jax
import jax.numpy as jnp
from jax import lax
from jax.experimental import pallas as pl
from jax.experimental.pallas import tpu as pltpu

_POOL = 7
_NREG = 32
_THR = 0.4
_NPAD = 1024
_N = 1000
_WIN = 64
_WINW = 72


def _nms_clip_kernel(rois_q_ref, rois_r_ref, rois_ref, out_ref):
    xs = rois_q_ref[0, 0]
    ys = rois_q_ref[0, 1]
    ws = rois_q_ref[0, 2]
    hs = rois_q_ref[0, 3]
    iota = lax.broadcasted_iota(jnp.int32, (8, 128), 0) * 128 + lax.broadcasted_iota(
        jnp.int32, (8, 128), 1
    )
    keep0 = jnp.where(iota < _N, 1.0, 0.0).astype(jnp.float32)

    def bcast(v11):
        return lax.broadcast_in_dim(v11, (8, 128), (0, 1))

    def body(p, keep):
        oh = iota == p
        kpv = bcast(jnp.max(jnp.where(oh, keep, 0.0), keepdims=True))
        row = rois_r_ref[0, p]
        xp = bcast(row[0:1, 0:1])
        yp = bcast(row[0:1, 1:2])
        wp = bcast(row[0:1, 2:3])
        hp = bcast(row[0:1, 3:4])
        x1 = jnp.maximum(xp, xs)
        y1 = jnp.maximum(yp, ys)
        x2 = jnp.minimum(xp + wp, xs + ws)
        y2 = jnp.minimum(yp + hp, ys + hs)
        inter = jnp.maximum(0.0, x2 - x1) * jnp.maximum(0.0, y2 - y1)
        union = wp * hp + ws * hs - inter
        iou = inter / jnp.maximum(union, 1e-9)
        suppress = jnp.logical_and(iou > _THR, iota > p)
        sup_f = jnp.where(suppress, 1.0, 0.0)
        return keep * (1.0 - sup_f * kpv)

    keep = lax.fori_loop(0, _N - 1, body, keep0)

    kiota = lax.broadcasted_iota(jnp.int32, (_NREG, 1), 0)

    def pick(k, carry):
        rem, idxc = carry
        cand = jnp.where(rem > 0.0, iota, _NPAD * 4)
        m = bcast(jnp.min(cand, keepdims=True))
        idx_k = lax.broadcast_in_dim(
            jnp.minimum(jnp.min(cand, keepdims=True), _N - 1), (_NREG, 1), (0, 1)
        )
        rem = jnp.where(iota == m, 0.0, rem)
        idxc = jnp.where(kiota == k, idx_k, idxc)
        return rem, idxc

    _, idxc = lax.fori_loop(
        0, _NREG, pick, (keep, jnp.zeros((_NREG, 1), jnp.int32))
    )

    iota_b = lax.broadcasted_iota(jnp.int32, (_NREG, _NPAD), 1)
    onehot = (idxc == iota_b).astype(jnp.float32)
    boxes = jnp.dot(
        onehot,
        rois_ref[0],
        preferred_element_type=jnp.float32,
        precision=lax.Precision.HIGHEST,
    )

    x = boxes[:, 0:1]
    y = boxes[:, 1:2]
    w = boxes[:, 2:3]
    h = boxes[:, 3:4]
    fs = 224
    c0 = jnp.maximum(0, x.astype(jnp.int32))
    c1 = jnp.maximum(0, y.astype(jnp.int32))
    c2 = jnp.minimum(fs, (x + w).astype(jnp.int32))
    c3 = jnp.minimum(fs, (y + h).astype(jnp.int32))

    def fix_dim(cmin, cmax):
        pad = _POOL - (cmax - cmin)
        fix_min = cmin < pad // 2
        fix_max = fs - cmax < (1 + pad) // 2
        pos = pad > 0
        sym = jnp.logical_and(pos, jnp.logical_not(jnp.logical_or(fix_min, fix_max)))
        ncmin = jnp.where(sym, cmin - pad // 2, cmin)
        ncmax = jnp.where(sym, cmax + (1 + pad) // 2, cmax)
        m1 = jnp.logical_and(pos, fix_min)
        ncmin = jnp.where(m1, 0, ncmin)
        ncmax = jnp.where(m1, _POOL, ncmax)
        m2 = jnp.logical_and(pos, fix_max)
        ncmin = jnp.where(m2, fs - _POOL, ncmin)
        ncmax = jnp.where(m2, fs, ncmax)
        return ncmin, ncmax

    c0, c2 = fix_dim(c0, c2)
    c1, c3 = fix_dim(c1, c3)
    res = jnp.concatenate([c0, c1, c2 - c0, c3 - c1], axis=1)
    out_ref[...] = res[None, :, :]


_BIN = 16


def _pool_kernel(coords_ref, feat_ref, out_ref, slab_ref, sem_ref):
    b = pl.program_id(0)
    r = pl.program_id(1)
    H = feat_ref.shape[1]
    W = feat_ref.shape[2]
    g = b * _NREG + r
    total = pl.num_programs(0) * _NREG

    def issue(gg, buf):
        b2 = gg // _NREG
        r2 = gg % _NREG
        x = coords_ref[b2, r2, 0]
        y = coords_ref[b2, r2, 1]
        ysl2 = jnp.minimum(y, H - _WIN)
        xsl2 = jnp.minimum((x // 8) * 8, W - _WINW)
        pltpu.make_async_copy(
            feat_ref.at[b2, pl.ds(ysl2, _WIN), pl.ds(xsl2, _WINW), :],
            slab_ref.at[buf],
            sem_ref.at[buf],
        ).start()

    @pl.when(g == 0)
    def _():
        issue(0, 0)

    @pl.when(g + 1 < total)
    def _():
        issue(g + 1, (g + 1) % 2)

    buf = g % 2
    pltpu.make_async_copy(
        feat_ref.at[b, pl.ds(0, _WIN), pl.ds(0, _WINW), :],
        slab_ref.at[buf],
        sem_ref.at[buf],
    ).wait()

    x0 = coords_ref[b, r, 0]
    y0 = coords_ref[b, r, 1]
    w = coords_ref[b, r, 2]
    h = coords_ref[b, r, 3]
    ysl = jnp.minimum(y0, H - _WIN)
    xsl = jnp.minimum((x0 // 8) * 8, W - _WINW)
    oy = y0 - ysl
    ox = x0 - xsl
    hstep = h // _POOL
    wstep = w // _POOL
    neg = jnp.float32(-jnp.inf)
    riota = lax.broadcasted_iota(jnp.int32, (_BIN, 1, 1), 0)
    ciota = lax.broadcasted_iota(jnp.int32, (_WINW, 1), 0)
    for i in range(_POOL):
        ya = oy + i * hstep
        yb = oy + (i + 1) * hstep if i + 1 < _POOL else oy + h
        ya_c = jnp.minimum(ya, _WIN - _BIN)
        chunk = slab_ref[buf, pl.ds(ya_c, _BIN), :, :]
        absr = riota + ya_c
        rm = jnp.logical_and(absr >= ya, absr < yb)
        rowmax = jnp.max(jnp.where(rm, chunk, neg), axis=0)
        for j in range(_POOL):
            xa = ox + j * wstep
            xb = ox + (j + 1) * wstep if j + 1 < _POOL else ox + w
            cm = jnp.logical_and(ciota >= xa, ciota < xb)
            cell = jnp.max(jnp.where(cm, rowmax, neg), axis=0)
            out_ref[0, 0, i, j, :] = cell


def kernel(features, roi):
    B, N = roi.shape[0], roi.shape[1]
    H, W, C = features.shape[1], features.shape[2], features.shape[3]
    rois_p = jnp.pad(roi, ((0, 0), (0, _NPAD - N), (0, 0)))
    rois_q = jnp.transpose(rois_p, (0, 2, 1)).reshape(B, 4, 8, 128)

    coords = pl.pallas_call(
        _nms_clip_kernel,
        grid=(B,),
        in_specs=[
            pl.BlockSpec((1, 4, 8, 128), lambda b: (b, 0, 0, 0)),
            pl.BlockSpec((1, _NPAD, 1, 4), lambda b: (b, 0, 0, 0)),
            pl.BlockSpec((1, _NPAD, 4), lambda b: (b, 0, 0)),
        ],
        out_specs=pl.BlockSpec((1, _NREG, 4), lambda b: (b, 0, 0)),
        out_shape=jax.ShapeDtypeStruct((B, _NREG, 4), jnp.int32),
    )(rois_q, rois_p[:, :, None, :], rois_p)

    out = pl.pallas_call(
        _pool_kernel,
        grid=(B, _NREG),
        in_specs=[
            pl.BlockSpec(memory_space=pltpu.SMEM),
            pl.BlockSpec(memory_space=pl.ANY),
        ],
        out_specs=pl.BlockSpec(
            (1, 1, _POOL, _POOL, C), lambda b, r: (b, r, 0, 0, 0)
        ),
        out_shape=jax.ShapeDtypeStruct((B, _NREG, _POOL, _POOL, C), jnp.float32),
        scratch_shapes=[
            pltpu.VMEM((2, _WIN, _WINW, C), jnp.float32),
            pltpu.SemaphoreType.DMA((2,)),
        ],
    )(coords, features)
    return out


# speedup vs baseline: 38.9100x; 1.0048x over previous
import jax
import jax.numpy as jnp
from jax import lax
from jax.experimental import pallas as pl
from jax.experimental.pallas import tpu as pltpu

_POOL = 7
_NREG = 32
_THR = 0.4
_NPAD = 1024
_N = 1000
_WIN = 64
_WINW = 72


def _nms_clip_kernel(rois_q_ref, rois_r_ref, rois_ref, out_ref):
    xs = rois_q_ref[0, 0]
    ys = rois_q_ref[0, 1]
    ws = rois_q_ref[0, 2]
    hs = rois_q_ref[0, 3]
    iota = lax.broadcasted_iota(jnp.int32, (8, 128), 0) * 128 + lax.broadcasted_iota(
        jnp.int32, (8, 128), 1
    )
    keep0 = jnp.where(iota < _N, 1.0, 0.0).astype(jnp.float32)

    def bcast(v11):
        return lax.broadcast_in_dim(v11, (8, 128), (0, 1))

    def step(p, keep):
        oh = iota == p
        kpv = bcast(jnp.max(jnp.where(oh, keep, 0.0), keepdims=True))
        row = rois_r_ref[0, p]
        xp = bcast(row[0:1, 0:1])
        yp = bcast(row[0:1, 1:2])
        wp = bcast(row[0:1, 2:3])
        hp = bcast(row[0:1, 3:4])
        x1 = jnp.maximum(xp, xs)
        y1 = jnp.maximum(yp, ys)
        x2 = jnp.minimum(xp + wp, xs + ws)
        y2 = jnp.minimum(yp + hp, ys + hs)
        inter = jnp.maximum(0.0, x2 - x1) * jnp.maximum(0.0, y2 - y1)
        union = wp * hp + ws * hs - inter
        iou = inter / jnp.maximum(union, 1e-9)
        suppress = jnp.logical_and(iou > _THR, iota > p)
        sup_f = jnp.where(suppress, 1.0, 0.0)
        return keep * (1.0 - sup_f * kpv)

    def body(i, keep):
        p0 = i * 8
        for d in range(8):
            keep = step(p0 + d, keep)
        return keep

    keep = lax.fori_loop(0, _NPAD // 8 - 3, body, keep0)

    kiota = lax.broadcasted_iota(jnp.int32, (_NREG, 1), 0)

    def pick(k, carry):
        rem, idxc = carry
        cand = jnp.where(rem > 0.0, iota, _NPAD * 4)
        m = bcast(jnp.min(cand, keepdims=True))
        idx_k = lax.broadcast_in_dim(
            jnp.minimum(jnp.min(cand, keepdims=True), _N - 1), (_NREG, 1), (0, 1)
        )
        rem = jnp.where(iota == m, 0.0, rem)
        idxc = jnp.where(kiota == k, idx_k, idxc)
        return rem, idxc

    _, idxc = lax.fori_loop(
        0, _NREG, pick, (keep, jnp.zeros((_NREG, 1), jnp.int32))
    )

    iota_b = lax.broadcasted_iota(jnp.int32, (_NREG, _NPAD), 1)
    onehot = (idxc == iota_b).astype(jnp.float32)
    boxes = jnp.dot(
        onehot,
        rois_ref[0],
        preferred_element_type=jnp.float32,
        precision=lax.Precision.HIGHEST,
    )

    x = boxes[:, 0:1]
    y = boxes[:, 1:2]
    w = boxes[:, 2:3]
    h = boxes[:, 3:4]
    fs = 224
    c0 = jnp.maximum(0, x.astype(jnp.int32))
    c1 = jnp.maximum(0, y.astype(jnp.int32))
    c2 = jnp.minimum(fs, (x + w).astype(jnp.int32))
    c3 = jnp.minimum(fs, (y + h).astype(jnp.int32))

    def fix_dim(cmin, cmax):
        pad = _POOL - (cmax - cmin)
        fix_min = cmin < pad // 2
        fix_max = fs - cmax < (1 + pad) // 2
        pos = pad > 0
        sym = jnp.logical_and(pos, jnp.logical_not(jnp.logical_or(fix_min, fix_max)))
        ncmin = jnp.where(sym, cmin - pad // 2, cmin)
        ncmax = jnp.where(sym, cmax + (1 + pad) // 2, cmax)
        m1 = jnp.logical_and(pos, fix_min)
        ncmin = jnp.where(m1, 0, ncmin)
        ncmax = jnp.where(m1, _POOL, ncmax)
        m2 = jnp.logical_and(pos, fix_max)
        ncmin = jnp.where(m2, fs - _POOL, ncmin)
        ncmax = jnp.where(m2, fs, ncmax)
        return ncmin, ncmax

    c0, c2 = fix_dim(c0, c2)
    c1, c3 = fix_dim(c1, c3)
    res = jnp.concatenate([c0, c1, c2 - c0, c3 - c1], axis=1)
    out_ref[...] = res[None, :, :]


_BIN = 16


def _pool_kernel(coords_ref, feat_ref, out_ref, slab_ref, sem_ref):
    b = pl.program_id(0)
    r = pl.program_id(1)
    H = feat_ref.shape[1]
    W = feat_ref.shape[2]
    g = b * _NREG + r
    total = pl.num_programs(0) * _NREG

    def issue(gg, buf):
        b2 = gg // _NREG
        r2 = gg % _NREG
        x = coords_ref[b2, r2, 0]
        y = coords_ref[b2, r2, 1]
        ysl2 = jnp.minimum(y, H - _WIN)
        xsl2 = jnp.minimum((x // 8) * 8, W - _WINW)
        pltpu.make_async_copy(
            feat_ref.at[b2, pl.ds(ysl2, _WIN), pl.ds(xsl2, _WINW), :],
            slab_ref.at[buf],
            sem_ref.at[buf],
        ).start()

    @pl.when(g == 0)
    def _():
        issue(0, 0)

    @pl.when(g + 1 < total)
    def _():
        issue(g + 1, (g + 1) % 2)

    buf = g % 2
    pltpu.make_async_copy(
        feat_ref.at[b, pl.ds(0, _WIN), pl.ds(0, _WINW), :],
        slab_ref.at[buf],
        sem_ref.at[buf],
    ).wait()

    x0 = coords_ref[b, r, 0]
    y0 = coords_ref[b, r, 1]
    w = coords_ref[b, r, 2]
    h = coords_ref[b, r, 3]
    ysl = jnp.minimum(y0, H - _WIN)
    xsl = jnp.minimum((x0 // 8) * 8, W - _WINW)
    oy = y0 - ysl
    ox = x0 - xsl
    hstep = h // _POOL
    wstep = w // _POOL
    neg = jnp.float32(-jnp.inf)
    riota = lax.broadcasted_iota(jnp.int32, (_BIN, 1, 1), 0)
    ciota = lax.broadcasted_iota(jnp.int32, (_WINW, 1), 0)
    for i in range(_POOL):
        ya = oy + i * hstep
        yb = oy + (i + 1) * hstep if i + 1 < _POOL else oy + h
        ya_c = jnp.minimum(ya, _WIN - _BIN)
        chunk = slab_ref[buf, pl.ds(ya_c, _BIN), :, :]
        absr = riota + ya_c
        rm = jnp.logical_and(absr >= ya, absr < yb)
        rowmax = jnp.max(jnp.where(rm, chunk, neg), axis=0)
        for j in range(_POOL):
            xa = ox + j * wstep
            xb = ox + (j + 1) * wstep if j + 1 < _POOL else ox + w
            cm = jnp.logical_and(ciota >= xa, ciota < xb)
            cell = jnp.max(jnp.where(cm, rowmax, neg), axis=0)
            out_ref[0, 0, i, j, :] = cell


def kernel(features, roi):
    B, N = roi.shape[0], roi.shape[1]
    H, W, C = features.shape[1], features.shape[2], features.shape[3]
    rois_p = jnp.pad(roi, ((0, 0), (0, _NPAD - N), (0, 0)))
    rois_q = jnp.transpose(rois_p, (0, 2, 1)).reshape(B, 4, 8, 128)

    coords = pl.pallas_call(
        _nms_clip_kernel,
        grid=(B,),
        in_specs=[
            pl.BlockSpec((1, 4, 8, 128), lambda b: (b, 0, 0, 0)),
            pl.BlockSpec((1, _NPAD, 1, 4), lambda b: (b, 0, 0, 0)),
            pl.BlockSpec((1, _NPAD, 4), lambda b: (b, 0, 0)),
        ],
        out_specs=pl.BlockSpec((1, _NREG, 4), lambda b: (b, 0, 0)),
        out_shape=jax.ShapeDtypeStruct((B, _NREG, 4), jnp.int32),
    )(rois_q, rois_p[:, :, None, :], rois_p)

    out = pl.pallas_call(
        _pool_kernel,
        grid=(B, _NREG),
        in_specs=[
            pl.BlockSpec(memory_space=pltpu.SMEM),
            pl.BlockSpec(memory_space=pl.ANY),
        ],
        out_specs=pl.BlockSpec(
            (1, 1, _POOL, _POOL, C), lambda b, r: (b, r, 0, 0, 0)
        ),
        out_shape=jax.ShapeDtypeStruct((B, _NREG, _POOL, _POOL, C), jnp.float32),
        scratch_shapes=[
            pltpu.VMEM((2, _WIN, _WINW, C), jnp.float32),
            pltpu.SemaphoreType.DMA((2,)),
        ],
    )(coords, features)
    return out


# speedup vs baseline: 39.2074x; 1.0076x over previous
import jax
import jax.numpy as jnp
from jax import lax
from jax.experimental import pallas as pl
from jax.experimental.pallas import tpu as pltpu

_POOL = 7
_NREG = 32
_THR = 0.4
_NPAD = 1024
_N = 1000
_WIN = 64
_WINW = 72


def _nms_clip_kernel(rois_q_ref, rois_r_ref, rois_ref, out_ref):
    xs = rois_q_ref[0, 0]
    ys = rois_q_ref[0, 1]
    ws = rois_q_ref[0, 2]
    hs = rois_q_ref[0, 3]
    iota = lax.broadcasted_iota(jnp.int32, (8, 128), 0) * 128 + lax.broadcasted_iota(
        jnp.int32, (8, 128), 1
    )
    keep0 = jnp.where(iota < _N, 1.0, 0.0).astype(jnp.float32)

    def bcast(v11):
        return lax.broadcast_in_dim(v11, (8, 128), (0, 1))

    pwf = jnp.left_shift(1, iota % 8).astype(jnp.float32)

    def body(i, keep):
        p0 = i * 8
        win = jnp.logical_and(iota >= p0, iota < p0 + 8)
        kwin = jnp.sum(jnp.where(win, keep * pwf, 0.0), keepdims=True)
        b = kwin.astype(jnp.int32)
        rows = []
        srows = []
        for d in range(8):
            p = p0 + d
            row = rois_r_ref[0, p]
            xp = bcast(row[0:1, 0:1])
            yp = bcast(row[0:1, 1:2])
            wp = bcast(row[0:1, 2:3])
            hp = bcast(row[0:1, 3:4])
            x1 = jnp.maximum(xp, xs)
            y1 = jnp.maximum(yp, ys)
            x2 = jnp.minimum(xp + wp, xs + ws)
            y2 = jnp.minimum(yp + hp, ys + hs)
            inter = jnp.maximum(0.0, x2 - x1) * jnp.maximum(0.0, y2 - y1)
            union = wp * hp + ws * hs - inter
            iou = inter / jnp.maximum(union, 1e-9)
            supf = jnp.where(jnp.logical_and(iou > _THR, iota > p), 1.0, 0.0)
            rows.append(supf)
            srows.append(
                jnp.sum(jnp.where(win, supf * pwf, 0.0), keepdims=True).astype(
                    jnp.int32
                )
            )
        bits = []
        for d in range(8):
            bit = jnp.bitwise_and(lax.shift_right_logical(b, d), 1)
            bits.append(bit)
            b = jnp.bitwise_and(b, jnp.bitwise_not(srows[d] * bit))
        for d in range(8):
            bf = lax.broadcast_in_dim(bits[d].astype(jnp.float32), (8, 128), (0, 1))
            keep = keep * (1.0 - rows[d] * bf)
        return keep

    keep = lax.fori_loop(0, _NPAD // 8 - 3, body, keep0)

    kiota = lax.broadcasted_iota(jnp.int32, (_NREG, 1), 0)

    def pick(k, carry):
        rem, idxc = carry
        cand = jnp.where(rem > 0.0, iota, _NPAD * 4)
        m = bcast(jnp.min(cand, keepdims=True))
        idx_k = lax.broadcast_in_dim(
            jnp.minimum(jnp.min(cand, keepdims=True), _N - 1), (_NREG, 1), (0, 1)
        )
        rem = jnp.where(iota == m, 0.0, rem)
        idxc = jnp.where(kiota == k, idx_k, idxc)
        return rem, idxc

    _, idxc = lax.fori_loop(
        0, _NREG, pick, (keep, jnp.zeros((_NREG, 1), jnp.int32))
    )

    iota_b = lax.broadcasted_iota(jnp.int32, (_NREG, _NPAD), 1)
    onehot = (idxc == iota_b).astype(jnp.float32)
    boxes = jnp.dot(
        onehot,
        rois_ref[0],
        preferred_element_type=jnp.float32,
        precision=lax.Precision.HIGHEST,
    )

    x = boxes[:, 0:1]
    y = boxes[:, 1:2]
    w = boxes[:, 2:3]
    h = boxes[:, 3:4]
    fs = 224
    c0 = jnp.maximum(0, x.astype(jnp.int32))
    c1 = jnp.maximum(0, y.astype(jnp.int32))
    c2 = jnp.minimum(fs, (x + w).astype(jnp.int32))
    c3 = jnp.minimum(fs, (y + h).astype(jnp.int32))

    def fix_dim(cmin, cmax):
        pad = _POOL - (cmax - cmin)
        fix_min = cmin < pad // 2
        fix_max = fs - cmax < (1 + pad) // 2
        pos = pad > 0
        sym = jnp.logical_and(pos, jnp.logical_not(jnp.logical_or(fix_min, fix_max)))
        ncmin = jnp.where(sym, cmin - pad // 2, cmin)
        ncmax = jnp.where(sym, cmax + (1 + pad) // 2, cmax)
        m1 = jnp.logical_and(pos, fix_min)
        ncmin = jnp.where(m1, 0, ncmin)
        ncmax = jnp.where(m1, _POOL, ncmax)
        m2 = jnp.logical_and(pos, fix_max)
        ncmin = jnp.where(m2, fs - _POOL, ncmin)
        ncmax = jnp.where(m2, fs, ncmax)
        return ncmin, ncmax

    c0, c2 = fix_dim(c0, c2)
    c1, c3 = fix_dim(c1, c3)
    res = jnp.concatenate([c0, c1, c2 - c0, c3 - c1], axis=1)
    out_ref[...] = res[None, :, :]


_BIN = 16


def _pool_kernel(coords_ref, feat_ref, out_ref, slab_ref, sem_ref):
    b = pl.program_id(0)
    r = pl.program_id(1)
    H = feat_ref.shape[1]
    W = feat_ref.shape[2]
    g = b * _NREG + r
    total = pl.num_programs(0) * _NREG

    def issue(gg, buf):
        b2 = gg // _NREG
        r2 = gg % _NREG
        x = coords_ref[b2, r2, 0]
        y = coords_ref[b2, r2, 1]
        ysl2 = jnp.minimum(y, H - _WIN)
        xsl2 = jnp.minimum((x // 8) * 8, W - _WINW)
        pltpu.make_async_copy(
            feat_ref.at[b2, pl.ds(ysl2, _WIN), pl.ds(xsl2, _WINW), :],
            slab_ref.at[buf],
            sem_ref.at[buf],
        ).start()

    @pl.when(g == 0)
    def _():
        issue(0, 0)

    @pl.when(g + 1 < total)
    def _():
        issue(g + 1, (g + 1) % 2)

    buf = g % 2
    pltpu.make_async_copy(
        feat_ref.at[b, pl.ds(0, _WIN), pl.ds(0, _WINW), :],
        slab_ref.at[buf],
        sem_ref.at[buf],
    ).wait()

    x0 = coords_ref[b, r, 0]
    y0 = coords_ref[b, r, 1]
    w = coords_ref[b, r, 2]
    h = coords_ref[b, r, 3]
    ysl = jnp.minimum(y0, H - _WIN)
    xsl = jnp.minimum((x0 // 8) * 8, W - _WINW)
    oy = y0 - ysl
    ox = x0 - xsl
    hstep = h // _POOL
    wstep = w // _POOL
    neg = jnp.float32(-jnp.inf)
    riota = lax.broadcasted_iota(jnp.int32, (_BIN, 1, 1), 0)
    ciota = lax.broadcasted_iota(jnp.int32, (_WINW, 1), 0)
    for i in range(_POOL):
        ya = oy + i * hstep
        yb = oy + (i + 1) * hstep if i + 1 < _POOL else oy + h
        ya_c = jnp.minimum(ya, _WIN - _BIN)
        chunk = slab_ref[buf, pl.ds(ya_c, _BIN), :, :]
        absr = riota + ya_c
        rm = jnp.logical_and(absr >= ya, absr < yb)
        rowmax = jnp.max(jnp.where(rm, chunk, neg), axis=0)
        for j in range(_POOL):
            xa = ox + j * wstep
            xb = ox + (j + 1) * wstep if j + 1 < _POOL else ox + w
            cm = jnp.logical_and(ciota >= xa, ciota < xb)
            cell = jnp.max(jnp.where(cm, rowmax, neg), axis=0)
            out_ref[0, 0, i, j, :] = cell


def kernel(features, roi):
    B, N = roi.shape[0], roi.shape[1]
    H, W, C = features.shape[1], features.shape[2], features.shape[3]
    rois_p = jnp.pad(roi, ((0, 0), (0, _NPAD - N), (0, 0)))
    rois_q = jnp.transpose(rois_p, (0, 2, 1)).reshape(B, 4, 8, 128)

    coords = pl.pallas_call(
        _nms_clip_kernel,
        grid=(B,),
        in_specs=[
            pl.BlockSpec((1, 4, 8, 128), lambda b: (b, 0, 0, 0)),
            pl.BlockSpec((1, _NPAD, 1, 4), lambda b: (b, 0, 0, 0)),
            pl.BlockSpec((1, _NPAD, 4), lambda b: (b, 0, 0)),
        ],
        out_specs=pl.BlockSpec((1, _NREG, 4), lambda b: (b, 0, 0)),
        out_shape=jax.ShapeDtypeStruct((B, _NREG, 4), jnp.int32),
    )(rois_q, rois_p[:, :, None, :], rois_p)

    out = pl.pallas_call(
        _pool_kernel,
        grid=(B, _NREG),
        in_specs=[
            pl.BlockSpec(memory_space=pltpu.SMEM),
            pl.BlockSpec(memory_space=pl.ANY),
        ],
        out_specs=pl.BlockSpec(
            (1, 1, _POOL, _POOL, C), lambda b, r: (b, r, 0, 0, 0)
        ),
        out_shape=jax.ShapeDtypeStruct((B, _NREG, _POOL, _POOL, C), jnp.float32),
        scratch_shapes=[
            pltpu.VMEM((2, _WIN, _WINW, C), jnp.float32),
            pltpu.SemaphoreType.DMA((2,)),
        ],
    )(coords, features)
    return out
